# baseline (speedup 1.0000x reference)
import jax
import jax.numpy as jnp
from jax import lax
from jax.experimental import pallas as pl
from jax.experimental.pallas import tpu as pltpu
from jax.experimental.pallas import tpu_sc as plsc

N = 320000
NT = 17
R = 8
D = 128
L = 16
NC, NS = 2, 16
NW = NC * NS
ROWS_PER_W = N // NW
C = 80
NCHUNK = ROWS_PER_W // C
GPC = C // L

NPAIR = 8
TS = D // 2
TBL_ROWS = NPAIR * 64 + R
NLOOK = NPAIR + 1


def _sc_body(w_hbm, x_hbm, out_hbm, wbuf, tbl,
             xb0, xb1, ob0, ob1, si0, si1, so0, so1):
    wid = lax.axis_index("s") * NC + lax.axis_index("c")
    base = wid * ROWS_PER_W

    pltpu.sync_copy(w_hbm, wbuf)

    iota = lax.iota(jnp.int32, L)

    def to_bf16_bits(v):
        w = lax.bitcast_convert_type(v, jnp.int32)
        rounded = w + 0x7FFF + ((w >> 16) & 1)
        return (rounded >> 16) & 0xFFFF

    def pack_row_to(vs, dst):
        for k in range(4):
            lo = to_bf16_bits(vs[2 * k])
            hi = to_bf16_bits(vs[2 * k + 1])
            word = lax.bitcast_convert_type(lo | (hi << 16), jnp.float32)
            tbl[pl.ds(dst + k * L, L)] = word

    def build_pair(p, _):
        def build_ab(ab, _):
            a = ab // R
            b = ab - a * R
            src_a = ((2 * p) * R + a) * D
            src_b = ((2 * p + 1) * R + b) * D
            vs = [wbuf[pl.ds(src_a + k * L, L)] + wbuf[pl.ds(src_b + k * L, L)]
                  for k in range(8)]
            pack_row_to(vs, (p * 64 + ab) * TS)
            return 0
        lax.fori_loop(0, 64, build_ab, 0)
        return 0
    lax.fori_loop(0, NPAIR, build_pair, 0)

    def build_single(r, _):
        src = (16 * R + r) * D
        vs = [wbuf[pl.ds(src + k * L, L)] for k in range(8)]
        pack_row_to(vs, (NPAIR * 64 + r) * TS)
        return 0
    lax.fori_loop(0, R, build_single, 0)

    xcol_base = iota * NT

    def in_copy(j, xb, sem):
        return pltpu.make_async_copy(
            x_hbm.at[pl.ds((base + j * C) * NT, C * NT)], xb, sem)

    def out_copy(j, ob, sem):
        return pltpu.make_async_copy(
            ob, out_hbm.at[pl.ds((base + j * C) * D, C * D)], sem)

    def compute(xb, obuf):
        def group_body(g, _):
            r0 = g * L
            xidx = xcol_base + r0 * NT
            cols = [plsc.load_gather(xb, [xidx + i]) for i in range(NT)]
            offs = []
            for p in range(NPAIR):
                offs.append((cols[2 * p] * R + cols[2 * p + 1] + p * 64) * TS)
            offs.append((cols[16] + NPAIR * 64) * TS)
            for r in range(L):
                osc = [off[r] for off in offs]
                ob = (r0 + r) * D
                for k in range(4):
                    acc = None
                    for t in range(NLOOK):
                        w = tbl[pl.ds(osc[t] + k * L, L)]
                        wb = plsc.bitcast(w, jnp.bfloat16)
                        acc = wb if acc is None else acc + wb
                    gi = plsc.bitcast(acc, jnp.int32)
                    lo = lax.bitcast_convert_type(gi << 16, jnp.float32)
                    hi = lax.bitcast_convert_type(gi & jnp.int32(-65536),
                                                  jnp.float32)
                    obuf[pl.ds(ob + k * 32, L)] = lo
                    obuf[pl.ds(ob + k * 32 + L, L)] = hi
            return 0
        lax.fori_loop(0, GPC, group_body, 0)

    in_copy(0, xb0, si0).start()

    def pair_body(jp, _):
        j0 = 2 * jp
        in_copy(j0, xb0, si0).wait()
        in_copy(j0 + 1, xb1, si1).start()

        @pl.when(jp > 0)
        def _wait0():
            out_copy(j0 - 2, ob0, so0).wait()
        compute(xb0, ob0)
        out_copy(j0, ob0, so0).start()

        in_copy(j0 + 1, xb1, si1).wait()
        in_copy(j0 + 2, xb0, si0).start()

        @pl.when(jp > 0)
        def _wait1():
            out_copy(j0 - 1, ob1, so1).wait()
        compute(xb1, ob1)
        out_copy(j0 + 1, ob1, so1).start()
        return 0

    lax.fori_loop(0, (NCHUNK - 1) // 2, pair_body, 0)

    jl = NCHUNK - 1
    in_copy(jl, xb0, si0).wait()
    out_copy(jl - 2, ob0, so0).wait()
    compute(xb0, ob0)
    out_copy(jl, ob0, so0).start()
    out_copy(jl - 1, ob1, so1).wait()
    out_copy(jl, ob0, so0).wait()


@jax.jit
def _encode(x_flat, w_flat):
    mesh = plsc.VectorSubcoreMesh(
        core_axis_name="c", subcore_axis_name="s", num_cores=NC, num_subcores=NS)
    f = pl.kernel(
        _sc_body,
        out_type=jax.ShapeDtypeStruct((N * D,), jnp.float32),
        mesh=mesh,
        compiler_params=pltpu.CompilerParams(needs_layout_passes=False),
        scratch_types=[
            pltpu.VMEM((NT * R * D,), jnp.float32),
            pltpu.VMEM((TBL_ROWS * TS,), jnp.float32),
            pltpu.VMEM((C * NT,), jnp.int32),
            pltpu.VMEM((C * NT,), jnp.int32),
            pltpu.VMEM((C * D,), jnp.float32),
            pltpu.VMEM((C * D,), jnp.float32),
            pltpu.SemaphoreType.DMA,
            pltpu.SemaphoreType.DMA,
            pltpu.SemaphoreType.DMA,
            pltpu.SemaphoreType.DMA,
        ],
    )
    return f(w_flat, x_flat)


def kernel(x, W):
    x_flat = x.reshape(-1).astype(jnp.int32)
    w_flat = W.reshape(-1)
    out = _encode(x_flat, w_flat)
    return out.reshape(N, D)


# speedup vs baseline: 1.4012x; 1.4012x over previous
import jax
import jax.numpy as jnp
from jax import lax
from jax.experimental import pallas as pl
from jax.experimental.pallas import tpu as pltpu
from jax.experimental.pallas import tpu_sc as plsc

N = 320000
NT = 17
R = 8
D = 128
L = 16
NC, NS = 2, 16
NW = NC * NS
ROWS_PER_W = N // NW
C = 400
NCHUNK = ROWS_PER_W // C
GPC = C // L

NPAIR = 8
TS = D // 2
TBL_ROWS = NPAIR * 64 + R
NLOOK = NPAIR + 1


def _sc_body(w_hbm, x_hbm, out_hbm, wbuf, tbl, xb0, ob0):
    wid = lax.axis_index("s") * NC + lax.axis_index("c")
    base = wid * ROWS_PER_W

    pltpu.sync_copy(w_hbm, wbuf)

    iota = lax.iota(jnp.int32, L)

    def to_bf16_bits(v):
        w = lax.bitcast_convert_type(v, jnp.int32)
        rounded = w + 0x7FFF + ((w >> 16) & 1)
        return (rounded >> 16) & 0xFFFF

    def pack_row_to(vs, dst):
        for k in range(4):
            lo = to_bf16_bits(vs[2 * k])
            hi = to_bf16_bits(vs[2 * k + 1])
            word = lax.bitcast_convert_type(lo | (hi << 16), jnp.float32)
            tbl[pl.ds(dst + k * L, L)] = word

    def build_pair(p, _):
        def build_ab(ab, _):
            a = ab // R
            b = ab - a * R
            src_a = ((2 * p) * R + a) * D
            src_b = ((2 * p + 1) * R + b) * D
            vs = [wbuf[pl.ds(src_a + k * L, L)] + wbuf[pl.ds(src_b + k * L, L)]
                  for k in range(8)]
            pack_row_to(vs, (p * 64 + ab) * TS)
            return 0
        lax.fori_loop(0, 64, build_ab, 0)
        return 0
    lax.fori_loop(0, NPAIR, build_pair, 0)

    def build_single(r, _):
        src = (16 * R + r) * D
        vs = [wbuf[pl.ds(src + k * L, L)] for k in range(8)]
        pack_row_to(vs, (NPAIR * 64 + r) * TS)
        return 0
    lax.fori_loop(0, R, build_single, 0)

    xcol_base = iota * NT

    def compute(xb, obuf):
        def group_body(g, _):
            r0 = g * L
            xidx = xcol_base + r0 * NT
            cols = [plsc.load_gather(xb, [xidx + i]) for i in range(NT)]
            offs = []
            for p in range(NPAIR):
                offs.append((cols[2 * p] * R + cols[2 * p + 1] + p * 64) * TS)
            offs.append((cols[16] + NPAIR * 64) * TS)
            for r in range(L):
                osc = [off[r] for off in offs]
                ob = (r0 + r) * D
                for k in range(4):
                    acc = None
                    for t in range(NLOOK):
                        w = tbl[pl.ds(osc[t] + k * L, L)]
                        wb = plsc.bitcast(w, jnp.bfloat16)
                        acc = wb if acc is None else acc + wb
                    gi = plsc.bitcast(acc, jnp.int32)
                    lo = lax.bitcast_convert_type(gi << 16, jnp.float32)
                    hi = lax.bitcast_convert_type(gi & jnp.int32(-65536),
                                                  jnp.float32)
                    obuf[pl.ds(ob + k * 32, L)] = lo
                    obuf[pl.ds(ob + k * 32 + L, L)] = hi
            return 0
        lax.fori_loop(0, GPC, group_body, 0)

    def chunk_body(j, _):
        row0 = base + j * C
        pltpu.sync_copy(x_hbm.at[pl.ds(row0 * NT, C * NT)], xb0)
        compute(xb0, ob0)
        pltpu.sync_copy(ob0, out_hbm.at[pl.ds(row0 * D, C * D)])
        return 0
    lax.fori_loop(0, NCHUNK, chunk_body, 0)


@jax.jit
def _encode(x_flat, w_flat):
    mesh = plsc.VectorSubcoreMesh(
        core_axis_name="c", subcore_axis_name="s", num_cores=NC, num_subcores=NS)
    f = pl.kernel(
        _sc_body,
        out_type=jax.ShapeDtypeStruct((N * D,), jnp.float32),
        mesh=mesh,
        compiler_params=pltpu.CompilerParams(needs_layout_passes=False),
        scratch_types=[
            pltpu.VMEM((NT * R * D,), jnp.float32),
            pltpu.VMEM((TBL_ROWS * TS,), jnp.float32),
            pltpu.VMEM((C * NT,), jnp.int32),
            pltpu.VMEM((C * D,), jnp.float32),
        ],
    )
    return f(w_flat, x_flat)


def kernel(x, W):
    x_flat = x.reshape(-1).astype(jnp.int32)
    w_flat = W.reshape(-1)
    out = _encode(x_flat, w_flat)
    return out.reshape(N, D)


# speedup vs baseline: 1.4523x; 1.0365x over previous
import jax
import jax.numpy as jnp
from jax import lax
from jax.experimental import pallas as pl
from jax.experimental.pallas import tpu as pltpu
from jax.experimental.pallas import tpu_sc as plsc

N = 320000
NT = 17
R = 8
D = 128
L = 16
NC, NS = 2, 16
NW = NC * NS
ROWS_PER_W = N // NW
C = 400
NCHUNK = ROWS_PER_W // C
GPC = C // L

NPAIR = 7
TS = D // 2
TRI0 = NPAIR * 64
TBL_ROWS = TRI0 + 512
NLOOK = NPAIR + 1


def _sc_body(w_hbm, x_hbm, out_hbm, tbl, xb0, ob0):
    wid = lax.axis_index("s") * NC + lax.axis_index("c")
    base = wid * ROWS_PER_W

    wbuf = ob0
    pltpu.sync_copy(w_hbm, wbuf.at[pl.ds(0, NT * R * D)])

    iota = lax.iota(jnp.int32, L)

    def to_bf16_bits(v):
        w = lax.bitcast_convert_type(v, jnp.int32)
        rounded = w + 0x7FFF + ((w >> 16) & 1)
        return (rounded >> 16) & 0xFFFF

    def pack_row_to(vs, dst):
        for k in range(4):
            lo = to_bf16_bits(vs[2 * k])
            hi = to_bf16_bits(vs[2 * k + 1])
            word = lax.bitcast_convert_type(lo | (hi << 16), jnp.float32)
            tbl[pl.ds(dst + k * L, L)] = word

    def build_pair(p, _):
        def build_ab(ab, _):
            a = ab // R
            b = ab - a * R
            src_a = ((2 * p) * R + a) * D
            src_b = ((2 * p + 1) * R + b) * D
            vs = [wbuf[pl.ds(src_a + k * L, L)] + wbuf[pl.ds(src_b + k * L, L)]
                  for k in range(8)]
            pack_row_to(vs, (p * 64 + ab) * TS)
            return 0
        lax.fori_loop(0, 64, build_ab, 0)
        return 0
    lax.fori_loop(0, NPAIR, build_pair, 0)

    def build_triple(abc, _):
        a = abc // 64
        bc = abc - a * 64
        b = bc // R
        c = bc - b * R
        src_a = (14 * R + a) * D
        src_b = (15 * R + b) * D
        src_c = (16 * R + c) * D
        vs = [wbuf[pl.ds(src_a + k * L, L)] + wbuf[pl.ds(src_b + k * L, L)]
              + wbuf[pl.ds(src_c + k * L, L)] for k in range(8)]
        pack_row_to(vs, (TRI0 + abc) * TS)
        return 0
    lax.fori_loop(0, 512, build_triple, 0)

    xcol_base = iota * NT

    def compute(xb, obuf):
        def group_body(g, _):
            r0 = g * L
            xidx = xcol_base + r0 * NT
            cols = [plsc.load_gather(xb, [xidx + i]) for i in range(NT)]
            offs = []
            for p in range(NPAIR):
                offs.append((cols[2 * p] * R + cols[2 * p + 1] + p * 64) * TS)
            offs.append((cols[14] * 64 + cols[15] * R + cols[16] + TRI0) * TS)
            for r in range(L):
                osc = [off[r] for off in offs]
                ob = (r0 + r) * D
                for k in range(4):
                    acc = None
                    for t in range(NLOOK):
                        w = tbl[pl.ds(osc[t] + k * L, L)]
                        wb = plsc.bitcast(w, jnp.bfloat16)
                        acc = wb if acc is None else acc + wb
                    gi = plsc.bitcast(acc, jnp.int32)
                    lo = lax.bitcast_convert_type(gi << 16, jnp.float32)
                    hi = lax.bitcast_convert_type(gi & jnp.int32(-65536),
                                                  jnp.float32)
                    obuf[pl.ds(ob + k * 32, L)] = lo
                    obuf[pl.ds(ob + k * 32 + L, L)] = hi
            return 0
        lax.fori_loop(0, GPC, group_body, 0)

    def chunk_body(j, _):
        row0 = base + j * C
        pltpu.sync_copy(x_hbm.at[pl.ds(row0 * NT, C * NT)], xb0)
        compute(xb0, ob0)
        pltpu.sync_copy(ob0, out_hbm.at[pl.ds(row0 * D, C * D)])
        return 0
    lax.fori_loop(0, NCHUNK, chunk_body, 0)


@jax.jit
def _encode(x_flat, w_flat):
    mesh = plsc.VectorSubcoreMesh(
        core_axis_name="c", subcore_axis_name="s", num_cores=NC, num_subcores=NS)
    f = pl.kernel(
        _sc_body,
        out_type=jax.ShapeDtypeStruct((N * D,), jnp.float32),
        mesh=mesh,
        compiler_params=pltpu.CompilerParams(needs_layout_passes=False),
        scratch_types=[
            pltpu.VMEM((TBL_ROWS * TS,), jnp.float32),
            pltpu.VMEM((C * NT,), jnp.int32),
            pltpu.VMEM((C * D,), jnp.float32),
        ],
    )
    return f(w_flat, x_flat)


def kernel(x, W):
    x_flat = x.reshape(-1).astype(jnp.int32)
    w_flat = W.reshape(-1)
    out = _encode(x_flat, w_flat)
    return out.reshape(N, D)


# speedup vs baseline: 2.1854x; 1.5048x over previous
import jax
import jax.numpy as jnp
from jax import lax
from jax.experimental import pallas as pl
from jax.experimental.pallas import tpu as pltpu
from jax.experimental.pallas import tpu_sc as plsc

N = 320000
NT = 17
R = 8
D = 128
L = 16
NC, NS = 2, 16
NW = NC * NS
ROWS_PER_W = N // NW
C = 400
NCHUNK = ROWS_PER_W // C
GPC = C // L

NPAIR = 7
TS = D // 2
TRI0 = NPAIR * 64
TBL_ROWS = TRI0 + 512
NLOOK = NPAIR + 1


def _sc_body(w_hbm, x_hbm, out_hbm, tbl, xb0, ob0):
    wid = lax.axis_index("s") * NC + lax.axis_index("c")
    base = wid * ROWS_PER_W

    wbuf = ob0
    pltpu.sync_copy(w_hbm, wbuf.at[pl.ds(0, NT * R * D)])

    iota = lax.iota(jnp.int32, L)

    def to_bf16_bits(v):
        w = lax.bitcast_convert_type(v, jnp.int32)
        rounded = w + 0x7FFF + ((w >> 16) & 1)
        return (rounded >> 16) & 0xFFFF

    def pack_row_to(vs, dst):
        for k in range(4):
            lo = to_bf16_bits(vs[2 * k])
            hi = to_bf16_bits(vs[2 * k + 1])
            word = lax.bitcast_convert_type(lo | (hi << 16), jnp.float32)
            tbl[pl.ds(dst + k * L, L)] = word

    def build_pair(p, _):
        def build_ab(ab, _):
            a = ab // R
            b = ab - a * R
            src_a = ((2 * p) * R + a) * D
            src_b = ((2 * p + 1) * R + b) * D
            vs = [wbuf[pl.ds(src_a + k * L, L)] + wbuf[pl.ds(src_b + k * L, L)]
                  for k in range(8)]
            pack_row_to(vs, (p * 64 + ab) * TS)
            return 0
        lax.fori_loop(0, 64, build_ab, 0)
        return 0
    lax.fori_loop(0, NPAIR, build_pair, 0)

    def build_triple(abc, _):
        a = abc // 64
        bc = abc - a * 64
        b = bc // R
        c = bc - b * R
        src_a = (14 * R + a) * D
        src_b = (15 * R + b) * D
        src_c = (16 * R + c) * D
        vs = [wbuf[pl.ds(src_a + k * L, L)] + wbuf[pl.ds(src_b + k * L, L)]
              + wbuf[pl.ds(src_c + k * L, L)] for k in range(8)]
        pack_row_to(vs, (TRI0 + abc) * TS)
        return 0
    lax.fori_loop(0, 512, build_triple, 0)

    xcol_base = iota * NT

    def compute(xb, obuf):
        def group_body(g, _):
            r0 = g * L
            xidx = xcol_base + r0 * NT
            cols = [plsc.load_gather(xb, [xidx + i]) for i in range(NT)]
            offs = []
            for p in range(NPAIR):
                offs.append((cols[2 * p] * R + cols[2 * p + 1] + p * 64) * TS)
            offs.append((cols[14] * 64 + cols[15] * R + cols[16] + TRI0) * TS)
            def loads(osc, k):
                return [plsc.bitcast(tbl[pl.ds(osc[t] + k * L, L)],
                                     jnp.bfloat16) for t in range(NLOOK)]

            def reduce_store(ws, ob, k):
                while len(ws) > 1:
                    ws = ([ws[i] + ws[i + 1]
                           for i in range(0, len(ws) - 1, 2)]
                          + ([ws[-1]] if len(ws) % 2 else []))
                gi = plsc.bitcast(ws[0], jnp.int32)
                lo = lax.bitcast_convert_type(gi << 16, jnp.float32)
                hi = lax.bitcast_convert_type(gi & jnp.int32(-65536),
                                              jnp.float32)
                obuf[pl.ds(ob + k * 32, L)] = lo
                obuf[pl.ds(ob + k * 32 + L, L)] = hi

            pending = None
            for r in range(L):
                osc = [off[r] for off in offs]
                ob = (r0 + r) * D
                for k in range(4):
                    ws = loads(osc, k)
                    if pending is not None:
                        reduce_store(*pending)
                    pending = (ws, ob, k)
            reduce_store(*pending)
            return 0
        lax.fori_loop(0, GPC, group_body, 0)

    def chunk_body(j, _):
        row0 = base + j * C
        pltpu.sync_copy(x_hbm.at[pl.ds(row0 * NT, C * NT)], xb0)
        compute(xb0, ob0)
        pltpu.sync_copy(ob0, out_hbm.at[pl.ds(row0 * D, C * D)])
        return 0
    lax.fori_loop(0, NCHUNK, chunk_body, 0)


@jax.jit
def _encode(x_flat, w_flat):
    mesh = plsc.VectorSubcoreMesh(
        core_axis_name="c", subcore_axis_name="s", num_cores=NC, num_subcores=NS)
    f = pl.kernel(
        _sc_body,
        out_type=jax.ShapeDtypeStruct((N * D,), jnp.float32),
        mesh=mesh,
        compiler_params=pltpu.CompilerParams(needs_layout_passes=False),
        scratch_types=[
            pltpu.VMEM((TBL_ROWS * TS,), jnp.float32),
            pltpu.VMEM((C * NT,), jnp.int32),
            pltpu.VMEM((C * D,), jnp.float32),
        ],
    )
    return f(w_flat, x_flat)


def kernel(x, W):
    x_flat = x.reshape(-1).astype(jnp.int32)
    w_flat = W.reshape(-1)
    out = _encode(x_flat, w_flat)
    return out.reshape(N, D)


# speedup vs baseline: 2.7223x; 1.2457x over previous
import jax
import jax.numpy as jnp
from jax import lax
from jax.experimental import pallas as pl
from jax.experimental.pallas import tpu as pltpu
from jax.experimental.pallas import tpu_sc as plsc

N = 320000
NT = 17
R = 8
D = 128
L = 16
NC, NS = 2, 16
NW = NC * NS
ROWS_PER_W = N // NW
C = 80
NCHUNK = ROWS_PER_W // C
GPC = C // L

NPAIR = 7
TS = D // 2
TRI0 = NPAIR * 64
TBL_ROWS = TRI0 + 512
NLOOK = NPAIR + 1


def _sc_body(w_hbm, x_hbm, out_hbm, tbl, xb0, ob0, si0, si1, so0, so1):
    wid = lax.axis_index("s") * NC + lax.axis_index("c")
    base = wid * ROWS_PER_W

    wbuf = ob0
    pltpu.sync_copy(w_hbm, wbuf.at[pl.ds(0, NT * R * D)])

    iota = lax.iota(jnp.int32, L)

    def to_bf16_bits(v):
        w = lax.bitcast_convert_type(v, jnp.int32)
        rounded = w + 0x7FFF + ((w >> 16) & 1)
        return (rounded >> 16) & 0xFFFF

    def pack_row_to(vs, dst):
        for k in range(4):
            lo = to_bf16_bits(vs[2 * k])
            hi = to_bf16_bits(vs[2 * k + 1])
            word = lax.bitcast_convert_type(lo | (hi << 16), jnp.float32)
            tbl[pl.ds(dst + k * L, L)] = word

    def build_pair(p, _):
        def build_ab(ab, _):
            a = ab // R
            b = ab - a * R
            src_a = ((2 * p) * R + a) * D
            src_b = ((2 * p + 1) * R + b) * D
            vs = [wbuf[pl.ds(src_a + k * L, L)] + wbuf[pl.ds(src_b + k * L, L)]
                  for k in range(8)]
            pack_row_to(vs, (p * 64 + ab) * TS)
            return 0
        lax.fori_loop(0, 64, build_ab, 0)
        return 0
    lax.fori_loop(0, NPAIR, build_pair, 0)

    def build_triple(abc, _):
        a = abc // 64
        bc = abc - a * 64
        b = bc // R
        c = bc - b * R
        src_a = (14 * R + a) * D
        src_b = (15 * R + b) * D
        src_c = (16 * R + c) * D
        vs = [wbuf[pl.ds(src_a + k * L, L)] + wbuf[pl.ds(src_b + k * L, L)]
              + wbuf[pl.ds(src_c + k * L, L)] for k in range(8)]
        pack_row_to(vs, (TRI0 + abc) * TS)
        return 0
    lax.fori_loop(0, 512, build_triple, 0)

    xcol_base = iota * NT

    def compute(par):
        xoff = par * (C * NT)
        ooff = par * (C * D)

        def group_body(g, _):
            r0 = g * L
            xidx = xcol_base + (xoff + r0 * NT)
            cols = [plsc.load_gather(xb0, [xidx + i]) for i in range(NT)]
            offs = []
            for p in range(NPAIR):
                offs.append((cols[2 * p] * R + cols[2 * p + 1] + p * 64) * TS)
            offs.append((cols[14] * 64 + cols[15] * R + cols[16] + TRI0) * TS)
            def loads(osc, k):
                return [plsc.bitcast(tbl[pl.ds(osc[t] + k * L, L)],
                                     jnp.bfloat16) for t in range(NLOOK)]

            def reduce_store(ws, ob, k):
                while len(ws) > 1:
                    ws = ([ws[i] + ws[i + 1]
                           for i in range(0, len(ws) - 1, 2)]
                          + ([ws[-1]] if len(ws) % 2 else []))
                gi = plsc.bitcast(ws[0], jnp.int32)
                lo = lax.bitcast_convert_type(gi << 16, jnp.float32)
                hi = lax.bitcast_convert_type(gi & jnp.int32(-65536),
                                              jnp.float32)
                ob0[pl.ds(ob + k * 32, L)] = lo
                ob0[pl.ds(ob + k * 32 + L, L)] = hi

            pending = []
            for r in range(L):
                osc = [off[r] for off in offs]
                ob = ooff + (r0 + r) * D
                for k in range(4):
                    ws = loads(osc, k)
                    if len(pending) == 2:
                        reduce_store(*pending.pop(0))
                    pending.append((ws, ob, k))
            for pend in pending:
                reduce_store(*pend)
            return 0
        lax.fori_loop(0, GPC, group_body, 0)

    def in_copy(j, par, sem):
        return pltpu.make_async_copy(
            x_hbm.at[pl.ds((base + j * C) * NT, C * NT)],
            xb0.at[pl.ds(par * (C * NT), C * NT)], sem)

    def out_copy(j, par, sem):
        return pltpu.make_async_copy(
            ob0.at[pl.ds(par * (C * D), C * D)],
            out_hbm.at[pl.ds((base + j * C) * D, C * D)], sem)

    in_copy(0, 0, si0).start()

    def chunk_body(j, _):
        par = j % 2

        @pl.when(par == 0)
        def _w0():
            in_copy(j, 0, si0).wait()

        @pl.when(par == 1)
        def _w1():
            in_copy(j, 1, si1).wait()

        @pl.when((par == 0) & (j + 1 < NCHUNK))
        def _p0():
            in_copy(j + 1, 1, si1).start()

        @pl.when((par == 1) & (j + 1 < NCHUNK))
        def _p1():
            in_copy(j + 1, 0, si0).start()

        @pl.when((par == 0) & (j >= 2))
        def _o0():
            out_copy(j - 2, 0, so0).wait()

        @pl.when((par == 1) & (j >= 2))
        def _o1():
            out_copy(j - 2, 1, so1).wait()

        compute(par)

        @pl.when(par == 0)
        def _s0():
            out_copy(j, 0, so0).start()

        @pl.when(par == 1)
        def _s1():
            out_copy(j, 1, so1).start()
        return 0
    lax.fori_loop(0, NCHUNK, chunk_body, 0)

    out_copy(NCHUNK - 2, 1, so1).wait()
    out_copy(NCHUNK - 1, 0, so0).wait()


@jax.jit
def _encode(x_flat, w_flat):
    mesh = plsc.VectorSubcoreMesh(
        core_axis_name="c", subcore_axis_name="s", num_cores=NC, num_subcores=NS)
    f = pl.kernel(
        _sc_body,
        out_type=jax.ShapeDtypeStruct((N * D,), jnp.float32),
        mesh=mesh,
        compiler_params=pltpu.CompilerParams(needs_layout_passes=False),
        scratch_types=[
            pltpu.VMEM((TBL_ROWS * TS,), jnp.float32),
            pltpu.VMEM((2 * C * NT,), jnp.int32),
            pltpu.VMEM((2 * C * D,), jnp.float32),
            pltpu.SemaphoreType.DMA,
            pltpu.SemaphoreType.DMA,
            pltpu.SemaphoreType.DMA,
            pltpu.SemaphoreType.DMA,
        ],
    )
    return f(w_flat, x_flat)


def kernel(x, W):
    x_flat = x.reshape(-1).astype(jnp.int32)
    w_flat = W.reshape(-1)
    out = _encode(x_flat, w_flat)
    return out.reshape(N, D)
